# baseline (speedup 1.0000x reference)
import functools

import jax
import jax.numpy as jnp
from jax import lax
from jax.experimental import pallas as pl
from jax.experimental.pallas import tpu as pltpu
from jax.experimental.pallas import tpu_sc as plsc


def _gather_tokens(embed_w, idx_flat):
    V, D = embed_w.shape
    B = idx_flat.shape[0]
    info = plsc.get_sparse_core_info()
    num_workers = info.num_cores * info.num_subcores
    b_per_w = B // num_workers
    mesh = plsc.VectorSubcoreMesh(core_axis_name="c", subcore_axis_name="s")

    @functools.partial(
        pl.kernel,
        mesh=mesh,
        out_type=jax.ShapeDtypeStruct((B, D), jnp.float32),
        scratch_types=[
            pltpu.VMEM((b_per_w,), jnp.int32),
            pltpu.VMEM((b_per_w, D), jnp.float32),
            pltpu.SemaphoreType.DMA,
        ],
    )
    def gather_k(table_hbm, idx_hbm, out_hbm, idx_v, rows_v, sem):
        wid = lax.axis_index("s") * info.num_cores + lax.axis_index("c")
        base = wid * b_per_w
        pltpu.sync_copy(idx_hbm.at[pl.ds(base, b_per_w)], idx_v)
        pltpu.async_copy(table_hbm.at[idx_v], rows_v, sem).wait()
        pltpu.sync_copy(rows_v, out_hbm.at[pl.ds(base, b_per_w)])

    return gather_k(embed_w, idx_flat)


def _ln_matmul_t(x, w):
    M, K = x.shape
    V = w.shape[0]
    NT = 1024
    DEPTH = 2
    nb_full = V // NT
    tail = V - nb_full * NT

    DW = 2

    def w_copy(w_hbm, w_buf, w_sems, j):
        s = lax.rem(j, DW) if not isinstance(j, int) else j % DW
        return pltpu.make_async_copy(
            w_hbm.at[pl.ds(j * NT, NT)], w_buf.at[s], w_sems.at[s])

    def o_copy(o_hbm, o_buf, o_sems, j, s):
        return pltpu.make_async_copy(
            o_buf.at[s], o_hbm.at[pl.ds(j * NT, NT), 0], o_sems.at[s])

    def body(x_hbm, w_hbm, o_hbm, xnt_hbm, x_v, xnt_v, w_buf, wbf_buf, o_buf,
             x_sem, w_sems, o_sems, t_sem):
        pltpu.make_async_copy(x_hbm, x_v, x_sem).start()
        for s in range(DW):
            w_copy(w_hbm, w_buf, w_sems, s).start()
        pltpu.make_async_copy(x_hbm, x_v, x_sem).wait()

        xf = x_v[...]
        mu = jnp.mean(xf, axis=1, keepdims=True)
        var = jnp.mean((xf - mu) ** 2, axis=1, keepdims=True)
        xn = (xf - mu) * lax.rsqrt(var + 1e-5)
        xnt_v[...] = xn.astype(jnp.bfloat16).T
        pltpu.make_async_copy(xnt_v, xnt_hbm, t_sem).start()

        w_copy(w_hbm, w_buf, w_sems, 0).wait()
        wbf_buf[0] = w_buf[0].astype(jnp.bfloat16)

        def step(j, _):
            s = lax.rem(j, DEPTH)
            sn = lax.rem(j + 1, DEPTH)

            @pl.when(j + 1 < nb_full)
            def _():
                w_copy(w_hbm, w_buf, w_sems, j + 1).wait()
                wbf_buf[sn] = w_buf[lax.rem(j + 1, DW)].astype(jnp.bfloat16)

            @pl.when(j + 2 < nb_full)
            def _():
                w_copy(w_hbm, w_buf, w_sems, j + 2).start()

            @pl.when(j >= DEPTH)
            def _():
                o_copy(o_hbm, o_buf, o_sems, j - DEPTH, s).wait()

            res = lax.dot_general(
                wbf_buf[s], xnt_v[...],
                (((1,), (0,)), ((), ())),
                preferred_element_type=jnp.float32)
            o_buf[s] = res
            o_copy(o_hbm, o_buf, o_sems, j, s).start()
            return 0

        lax.fori_loop(0, nb_full, step, 0)
        pltpu.make_async_copy(xnt_v, xnt_hbm, t_sem).wait()
        for s in range(DEPTH):
            jj = nb_full - DEPTH + s
            o_copy(o_hbm, o_buf, o_sems, jj, lax.rem(jj, DEPTH)).wait()

    out, xnt = pl.pallas_call(
        body,
        in_specs=[
            pl.BlockSpec(memory_space=pltpu.HBM),
            pl.BlockSpec(memory_space=pltpu.HBM),
        ],
        out_specs=[
            pl.BlockSpec(memory_space=pltpu.HBM),
            pl.BlockSpec(memory_space=pltpu.HBM),
        ],
        out_shape=[
            jax.ShapeDtypeStruct((V, 1, M), jnp.float32),
            jax.ShapeDtypeStruct((K, M), jnp.bfloat16),
        ],
        scratch_shapes=[
            pltpu.VMEM((M, K), jnp.float32),
            pltpu.VMEM((K, M), jnp.bfloat16),
            pltpu.VMEM((2, NT, K), jnp.float32),
            pltpu.VMEM((DEPTH, NT, K), jnp.bfloat16),
            pltpu.VMEM((DEPTH, NT, M), jnp.float32),
            pltpu.SemaphoreType.DMA,
            pltpu.SemaphoreType.DMA((2,)),
            pltpu.SemaphoreType.DMA((DEPTH,)),
            pltpu.SemaphoreType.DMA,
        ],
    )(x, w)

    def tail_body(xnt_ref, w_ref, prev_ref, o_ref):
        wt = w_ref[...].astype(jnp.bfloat16)
        o_ref[:, 0, :] = lax.dot_general(
            wt, xnt_ref[...], (((1,), (0,)), ((), ())),
            preferred_element_type=jnp.float32)

    NTT = 128
    tb = (nb_full * NT) // NTT
    out = pl.pallas_call(
        tail_body,
        grid=(1,),
        in_specs=[
            pl.BlockSpec((K, M), lambda j: (0, 0)),
            pl.BlockSpec((NTT, K), lambda j: (tb, 0)),
            pl.BlockSpec(memory_space=pltpu.HBM),
        ],
        out_specs=pl.BlockSpec((NTT, 1, M), lambda j: (tb, 0, 0)),
        out_shape=jax.ShapeDtypeStruct((V, 1, M), jnp.float32),
        input_output_aliases={2: 0},
    )(xnt, w, out)
    return jnp.transpose(out, (1, 2, 0))


def kernel(idx, embed_w, lm_head_w):
    B, T = idx.shape
    x = _gather_tokens(embed_w, idx.reshape(-1))
    return _ln_matmul_t(x, lm_head_w)


# speedup vs baseline: 1.0588x; 1.0588x over previous
import functools

import jax
import jax.numpy as jnp
from jax import lax
from jax.experimental import pallas as pl
from jax.experimental.pallas import tpu as pltpu
from jax.experimental.pallas import tpu_sc as plsc


def _gather_tokens(embed_w, idx_flat):
    V, D = embed_w.shape
    B = idx_flat.shape[0]
    info = plsc.get_sparse_core_info()
    num_workers = info.num_cores * info.num_subcores
    b_per_w = B // num_workers
    mesh = plsc.VectorSubcoreMesh(core_axis_name="c", subcore_axis_name="s")

    @functools.partial(
        pl.kernel,
        mesh=mesh,
        out_type=jax.ShapeDtypeStruct((B, D), jnp.float32),
        scratch_types=[
            pltpu.VMEM((b_per_w,), jnp.int32),
            pltpu.VMEM((b_per_w, D), jnp.float32),
            pltpu.SemaphoreType.DMA,
        ],
    )
    def gather_k(table_hbm, idx_hbm, out_hbm, idx_v, rows_v, sem):
        wid = lax.axis_index("s") * info.num_cores + lax.axis_index("c")
        base = wid * b_per_w
        pltpu.sync_copy(idx_hbm.at[pl.ds(base, b_per_w)], idx_v)
        pltpu.async_copy(table_hbm.at[idx_v], rows_v, sem).wait()
        pltpu.sync_copy(rows_v, out_hbm.at[pl.ds(base, b_per_w)])

    return gather_k(embed_w, idx_flat)


def _ln_matmul_t(x, w):
    M, K = x.shape
    V = w.shape[0]
    NT = 1024
    DEPTH = 3
    nb_full = V // NT
    tail = V - nb_full * NT

    def w_copy(w_hbm, w_buf, w_sems, j, s):
        return pltpu.make_async_copy(
            w_hbm.at[pl.ds(j * NT, NT)], w_buf.at[s], w_sems.at[s])

    def o_copy(o_hbm, o_buf, o_sems, j, s):
        return pltpu.make_async_copy(
            o_buf.at[s], o_hbm.at[pl.ds(j * NT, NT), 0], o_sems.at[s])

    def body(x_hbm, w_hbm, o_hbm, xnt_hbm, x_v, xnt_v, w_buf, o_buf,
             x_sem, w_sems, o_sems, t_sem):
        pltpu.make_async_copy(x_hbm, x_v, x_sem).start()
        for s in range(DEPTH):
            w_copy(w_hbm, w_buf, w_sems, s, s).start()
        pltpu.make_async_copy(x_hbm, x_v, x_sem).wait()

        xf = x_v[...]
        mu = jnp.mean(xf, axis=1, keepdims=True)
        var = jnp.mean((xf - mu) ** 2, axis=1, keepdims=True)
        xn = (xf - mu) * lax.rsqrt(var + 1e-5)
        xnt_v[...] = xn.astype(jnp.bfloat16).T
        pltpu.make_async_copy(xnt_v, xnt_hbm, t_sem).start()

        def step(j, _):
            s = lax.rem(j, DEPTH)
            w_copy(w_hbm, w_buf, w_sems, j, s).wait()

            @pl.when(j >= DEPTH)
            def _():
                o_copy(o_hbm, o_buf, o_sems, j - DEPTH, s).wait()

            res = lax.dot_general(
                w_buf[s].astype(jnp.bfloat16), xnt_v[...],
                (((1,), (0,)), ((), ())),
                preferred_element_type=jnp.float32)
            o_buf[s] = res
            o_copy(o_hbm, o_buf, o_sems, j, s).start()

            @pl.when(j + DEPTH < nb_full)
            def _():
                w_copy(w_hbm, w_buf, w_sems, j + DEPTH, s).start()
            return 0

        lax.fori_loop(0, nb_full, step, 0)
        pltpu.make_async_copy(xnt_v, xnt_hbm, t_sem).wait()
        for s in range(DEPTH):
            jj = nb_full - DEPTH + s
            o_copy(o_hbm, o_buf, o_sems, jj, lax.rem(jj, DEPTH)).wait()

    out, xnt = pl.pallas_call(
        body,
        in_specs=[
            pl.BlockSpec(memory_space=pltpu.HBM),
            pl.BlockSpec(memory_space=pltpu.HBM),
        ],
        out_specs=[
            pl.BlockSpec(memory_space=pltpu.HBM),
            pl.BlockSpec(memory_space=pltpu.HBM),
        ],
        out_shape=[
            jax.ShapeDtypeStruct((V, 1, M), jnp.float32),
            jax.ShapeDtypeStruct((K, M), jnp.bfloat16),
        ],
        scratch_shapes=[
            pltpu.VMEM((M, K), jnp.float32),
            pltpu.VMEM((K, M), jnp.bfloat16),
            pltpu.VMEM((DEPTH, NT, K), jnp.float32),
            pltpu.VMEM((DEPTH, NT, M), jnp.float32),
            pltpu.SemaphoreType.DMA,
            pltpu.SemaphoreType.DMA((DEPTH,)),
            pltpu.SemaphoreType.DMA((DEPTH,)),
            pltpu.SemaphoreType.DMA,
        ],
    )(x, w)

    def tail_body(xnt_ref, w_ref, prev_ref, o_ref):
        wt = w_ref[...].astype(jnp.bfloat16)
        o_ref[:, 0, :] = lax.dot_general(
            wt, xnt_ref[...], (((1,), (0,)), ((), ())),
            preferred_element_type=jnp.float32)

    NTT = 128
    tb = (nb_full * NT) // NTT
    out = pl.pallas_call(
        tail_body,
        grid=(1,),
        in_specs=[
            pl.BlockSpec((K, M), lambda j: (0, 0)),
            pl.BlockSpec((NTT, K), lambda j: (tb, 0)),
            pl.BlockSpec(memory_space=pltpu.HBM),
        ],
        out_specs=pl.BlockSpec((NTT, 1, M), lambda j: (tb, 0, 0)),
        out_shape=jax.ShapeDtypeStruct((V, 1, M), jnp.float32),
        input_output_aliases={2: 0},
    )(xnt, w, out)
    return jnp.transpose(out, (1, 2, 0))


def kernel(idx, embed_w, lm_head_w):
    B, T = idx.shape
    x = _gather_tokens(embed_w, idx.reshape(-1))
    return _ln_matmul_t(x, lm_head_w)
